# baseline (speedup 1.0000x reference)
import functools

import jax
import jax.numpy as jnp
from jax import lax
from jax.experimental import pallas as pl
from jax.experimental.pallas import tpu as pltpu
from jax.experimental.pallas import tpu_sc as plsc

VOC_DIM = 64
NC, NS = 2, 16
NW = NC * NS
LANES = 16


def _gather_sum(B):
    bpw = B // NW
    mesh = plsc.VectorSubcoreMesh(core_axis_name="c", subcore_axis_name="s")

    @functools.partial(
        pl.kernel,
        out_type=jax.ShapeDtypeStruct((B, VOC_DIM), jnp.float32),
        mesh=mesh,
        compiler_params=pltpu.CompilerParams(use_tc_tiling_on_sc=False),
        scratch_types=[
            pltpu.VMEM((bpw,), jnp.int32),
            pltpu.VMEM((bpw,), jnp.int32),
            pltpu.VMEM((bpw,), jnp.int32),
            pltpu.VMEM((bpw,), jnp.int32),
            pltpu.VMEM((bpw, VOC_DIM), jnp.float32),
            pltpu.VMEM((bpw, VOC_DIM), jnp.float32),
            pltpu.VMEM((bpw, VOC_DIM), jnp.float32),
            pltpu.SemaphoreType.DMA,
            pltpu.SemaphoreType.DMA,
            pltpu.SemaphoreType.DMA,
        ],
    )
    def gather_sum(x1h, x2h, x4h, x5h, tableh, outh,
                   i1, i2, i4, i5, accv, bv, cv, s1, s2, s3):
        wid = lax.axis_index("s") * NC + lax.axis_index("c")
        base = wid * bpw
        sl = pl.ds(base, bpw)
        pltpu.sync_copy(x1h.at[sl], i1)
        pltpu.sync_copy(x2h.at[sl], i2)
        pltpu.sync_copy(x4h.at[sl], i4)
        pltpu.sync_copy(x5h.at[sl], i5)

        cp1 = pltpu.async_copy(tableh.at[i1], accv, s1)
        cp2 = pltpu.async_copy(tableh.at[i2], bv, s2)
        cp4 = pltpu.async_copy(tableh.at[i4], cv, s3)

        def add_rows(dst, src):
            def body(r, _):
                for c in range(VOC_DIM // LANES):
                    s = pl.ds(c * LANES, LANES)
                    plsc.addupdate(dst.at[r, s], src[r, s])
                return 0
            lax.fori_loop(0, bpw, body, 0)

        cp1.wait()
        cp2.wait()
        add_rows(accv, bv)
        cp4.wait()
        cp5 = pltpu.async_copy(tableh.at[i5], bv, s2)
        add_rows(accv, cv)
        cp5.wait()
        add_rows(accv, bv)
        pltpu.sync_copy(accv, outh.at[sl])

    return gather_sum


def _project(summed, W1):
    B = summed.shape[0]
    blk = 2048

    def body(x_ref, w_ref, o_ref):
        o_ref[...] = lax.dot_general(
            x_ref[...], w_ref[...], (((1,), (1,)), ((), ())),
            preferred_element_type=jnp.float32)

    return pl.pallas_call(
        body,
        grid=(B // blk,),
        in_specs=[
            pl.BlockSpec((blk, VOC_DIM), lambda i: (i, 0)),
            pl.BlockSpec((VOC_DIM, VOC_DIM), lambda i: (0, 0)),
        ],
        out_specs=pl.BlockSpec((blk, VOC_DIM), lambda i: (i, 0)),
        out_shape=jax.ShapeDtypeStruct((B, VOC_DIM), jnp.float32),
    )(summed, W1)


def kernel(x1, x2, x4, x5, codebook, W1):
    B = x1.shape[0]
    summed = _gather_sum(B)(x1, x2, x4, x5, codebook)
    return _project(summed, W1)


# speedup vs baseline: 1.6040x; 1.6040x over previous
import functools

import jax
import jax.numpy as jnp
from jax import lax
from jax.experimental import pallas as pl
from jax.experimental.pallas import tpu as pltpu
from jax.experimental.pallas import tpu_sc as plsc

VOC_DIM = 64
NC, NS = 2, 16
NW = NC * NS
LANES = 16


def _gather_sum(B):
    bpw = B // NW
    mesh = plsc.VectorSubcoreMesh(core_axis_name="c", subcore_axis_name="s")

    @functools.partial(
        pl.kernel,
        out_type=jax.ShapeDtypeStruct((B, VOC_DIM), jnp.float32),
        mesh=mesh,
        scratch_types=[
            pltpu.SMEM((bpw,), jnp.int32),
            pltpu.VMEM((bpw,), jnp.int32),
            pltpu.VMEM((bpw, VOC_DIM), jnp.float32),
            pltpu.VMEM((bpw, VOC_DIM), jnp.float32),
            pltpu.VMEM((bpw, VOC_DIM), jnp.float32),
            pltpu.SemaphoreType.DMA,
            pltpu.SemaphoreType.DMA,
            pltpu.SemaphoreType.DMA,
        ],
    )
    def gather_sum(x1h, x2h, x4h, x5h, tableh, outh,
                   idx_s, idx_v, accv, bv, cv, s1, s2, s3):
        wid = lax.axis_index("s") * NC + lax.axis_index("c")
        base = wid * bpw
        sl = pl.ds(base, bpw)

        def fire(xh, dstv, sem):
            pltpu.sync_copy(xh.at[sl], idx_v)

            def stage16(t, _):
                v = idx_v[pl.ds(t * LANES, LANES)]
                for lane in range(LANES):
                    idx_s[t * LANES + lane] = v[lane]
                return 0
            lax.fori_loop(0, bpw // LANES, stage16, 0)

            def body(j, _):
                i = idx_s[j]
                pltpu.async_copy(
                    tableh.at[pl.ds(i, 1), :], dstv.at[pl.ds(j, 1), :], sem)
                return 0
            lax.fori_loop(0, bpw, body, 0)

        def drain(dstv, sem):
            pltpu.make_async_copy(
                tableh.at[pl.ds(0, bpw), :], dstv, sem).wait()

        def add_rows(dst, src):
            def body(r, _):
                for c in range(VOC_DIM // LANES):
                    s = pl.ds(c * LANES, LANES)
                    plsc.addupdate(dst.at[r, s], src[r, s])
                return 0
            lax.fori_loop(0, bpw, body, 0)

        fire(x1h, accv, s1)
        fire(x2h, bv, s2)
        fire(x4h, cv, s3)
        drain(accv, s1)
        drain(bv, s2)
        add_rows(accv, bv)
        drain(cv, s3)
        fire(x5h, bv, s2)
        add_rows(accv, cv)
        drain(bv, s2)
        add_rows(accv, bv)
        pltpu.sync_copy(accv, outh.at[sl])

    return gather_sum


def _project(summed, W1):
    B = summed.shape[0]
    blk = 2048

    def body(x_ref, w_ref, o_ref):
        o_ref[...] = lax.dot_general(
            x_ref[...], w_ref[...], (((1,), (1,)), ((), ())),
            preferred_element_type=jnp.float32)

    return pl.pallas_call(
        body,
        grid=(B // blk,),
        in_specs=[
            pl.BlockSpec((blk, VOC_DIM), lambda i: (i, 0)),
            pl.BlockSpec((VOC_DIM, VOC_DIM), lambda i: (0, 0)),
        ],
        out_specs=pl.BlockSpec((blk, VOC_DIM), lambda i: (i, 0)),
        out_shape=jax.ShapeDtypeStruct((B, VOC_DIM), jnp.float32),
    )(summed, W1)


def kernel(x1, x2, x4, x5, codebook, W1):
    B = x1.shape[0]
    h = B // 2
    gs = _gather_sum(h)
    s0 = gs(x1[:h], x2[:h], x4[:h], x5[:h], codebook)
    s1 = gs(x1[h:], x2[h:], x4[h:], x5[h:], codebook)
    summed = jnp.concatenate([s0, s1], axis=0)
    return _project(summed, W1)
